# baseline (speedup 1.0000x reference)
import functools

import jax
import jax.numpy as jnp
from jax import lax
from jax.experimental import pallas as pl
from jax.experimental.pallas import tpu as pltpu
from jax.experimental.pallas import tpu_sc as plsc

N = 10000
E = 320000
H = 128
HH = H // 2
C = 16

NC = 2
NS = 16
NW = NC * NS
K = 125
NPAD = 10240
STRIPE = NPAD // NS
RB = 128

NCHUNK_CORE = E // NS // K
NCHUNK_EDGE = E // NW // K

_NOTILE = pltpu.CompilerParams(use_tc_tiling_on_sc=False)


def _sc_mesh():
    return plsc.VectorSubcoreMesh(core_axis_name="c", subcore_axis_name="s")


@functools.partial(
    pl.kernel,
    mesh=_sc_mesh(),
    out_type=jax.ShapeDtypeStruct((NC * 2 * NPAD,), jnp.float32),
    scratch_types=[
        pltpu.VMEM((NCHUNK_EDGE, K), jnp.int32),
        pltpu.VMEM((NCHUNK_EDGE, K), jnp.int32),
        pltpu.VMEM((K,), jnp.float32),
        pltpu.VMEM((STRIPE,), jnp.float32),
        pltpu.VMEM_SHARED((NPAD,), jnp.float32),
        pltpu.VMEM_SHARED((NPAD,), jnp.float32),
    ],
)
def _sc_degrees(src2d, dst2d, zeros1, ones_k, out, src_v, dst_v, ones_v,
                stage_v, deg_o, deg_i):
    c = lax.axis_index("c")
    s = lax.axis_index("s")
    wid = s * NC + c
    pltpu.sync_copy(zeros1, stage_v)
    pltpu.sync_copy(stage_v, deg_o.at[pl.ds(s * STRIPE, STRIPE)])
    pltpu.sync_copy(stage_v, deg_i.at[pl.ds(s * STRIPE, STRIPE)])
    plsc.subcore_barrier()
    pltpu.sync_copy(ones_k, ones_v)
    pltpu.sync_copy(src2d.at[pl.ds(wid * NCHUNK_EDGE, NCHUNK_EDGE)], src_v)
    pltpu.sync_copy(dst2d.at[pl.ds(wid * NCHUNK_EDGE, NCHUNK_EDGE)], dst_v)

    def chunk(j, carry):
        pltpu.sync_copy(ones_v, deg_o.at[src_v.at[j]], add=True)
        pltpu.sync_copy(ones_v, deg_i.at[dst_v.at[j]], add=True)
        return carry

    lax.fori_loop(0, NCHUNK_EDGE, chunk, 0)
    plsc.subcore_barrier()
    pltpu.sync_copy(deg_o.at[pl.ds(s * STRIPE, STRIPE)], stage_v)
    pltpu.sync_copy(stage_v, out.at[pl.ds((c * 2 + 0) * NPAD + s * STRIPE,
                                          STRIPE)])
    pltpu.sync_copy(deg_i.at[pl.ds(s * STRIPE, STRIPE)], stage_v)
    pltpu.sync_copy(stage_v, out.at[pl.ds((c * 2 + 1) * NPAD + s * STRIPE,
                                          STRIPE)])


@functools.partial(
    pl.kernel,
    mesh=_sc_mesh(),
    compiler_params=_NOTILE,
    out_type=jax.ShapeDtypeStruct((NC, NPAD, HH), jnp.float32),
    scratch_types=[
        pltpu.VMEM((NCHUNK_CORE, K), jnp.int32),
        pltpu.VMEM((NCHUNK_CORE, K), jnp.int32),
        [pltpu.VMEM((K, HH), jnp.float32)] * 4,
        pltpu.VMEM((RB, HH), jnp.float32),
        pltpu.VMEM_SHARED((NPAD, HH), jnp.float32),
        [pltpu.SemaphoreType.DMA] * 4,
        [pltpu.SemaphoreType.DMA] * 4,
    ],
)
def _sc_agg128(hp2, srcx, dst2d, zrows, out, src_v, dst_v, rows_v, stage_v,
               acc, gsem, ssem):
    c = lax.axis_index("c")
    s = lax.axis_index("s")
    row0 = s * STRIPE
    pltpu.sync_copy(zrows, stage_v)
    for r in range(STRIPE // RB):
        pltpu.sync_copy(stage_v, acc.at[pl.ds(row0 + r * RB, RB)])
    plsc.subcore_barrier()
    pltpu.sync_copy(srcx.at[c, pl.ds(s * NCHUNK_CORE, NCHUNK_CORE)], src_v)
    pltpu.sync_copy(dst2d.at[pl.ds(s * NCHUNK_CORE, NCHUNK_CORE)], dst_v)

    NB = 4
    for t in range(NB):
        pltpu.async_copy(hp2.at[src_v.at[t]], rows_v[t], gsem[t])

    def wave(q, carry):
        j = NB * q
        sc = []
        for t in range(NB):
            pltpu.make_async_copy(hp2.at[src_v.at[j + t]], rows_v[t],
                                  gsem[t]).wait()
            sc.append(pltpu.async_copy(rows_v[t], acc.at[dst_v.at[j + t]],
                                       ssem[t], add=True))
        for t in range(NB):
            sc[t].wait()
            pltpu.async_copy(hp2.at[src_v.at[j + NB + t]], rows_v[t], gsem[t])
        return carry

    lax.fori_loop(0, NCHUNK_CORE // NB - 1, wave, 0)
    jf = NCHUNK_CORE - NB
    sc = []
    for t in range(NB):
        pltpu.make_async_copy(hp2.at[src_v.at[jf + t]], rows_v[t],
                              gsem[t]).wait()
        sc.append(pltpu.async_copy(rows_v[t], acc.at[dst_v.at[jf + t]],
                                   ssem[t], add=True))
    for t in range(NB):
        sc[t].wait()
    plsc.subcore_barrier()
    for r in range(STRIPE // RB):
        pltpu.sync_copy(acc.at[pl.ds(row0 + r * RB, RB)], stage_v)
        pltpu.sync_copy(stage_v, out.at[c, pl.ds(row0 + r * RB, RB)])


@functools.partial(
    pl.kernel,
    mesh=_sc_mesh(),
    compiler_params=_NOTILE,
    out_type=jax.ShapeDtypeStruct((NC, NPAD, C), jnp.float32),
    scratch_types=[
        pltpu.VMEM((NCHUNK_EDGE, K), jnp.int32),
        pltpu.VMEM((NCHUNK_EDGE, K), jnp.int32),
        [pltpu.VMEM((K, C), jnp.float32)] * 4,
        pltpu.VMEM((RB, C), jnp.float32),
        pltpu.VMEM_SHARED((NPAD, C), jnp.float32),
        [pltpu.SemaphoreType.DMA] * 4,
        [pltpu.SemaphoreType.DMA] * 4,
    ],
)
def _sc_agg16(hp, src2d, dst2d, zrows, out, src_v, dst_v, rows_v, stage_v,
              acc, gsem, ssem):
    c = lax.axis_index("c")
    s = lax.axis_index("s")
    wid = s * NC + c
    row0 = s * STRIPE
    pltpu.sync_copy(zrows, stage_v)
    for r in range(STRIPE // RB):
        pltpu.sync_copy(stage_v, acc.at[pl.ds(row0 + r * RB, RB)])
    plsc.subcore_barrier()
    pltpu.sync_copy(src2d.at[pl.ds(wid * NCHUNK_EDGE, NCHUNK_EDGE)], src_v)
    pltpu.sync_copy(dst2d.at[pl.ds(wid * NCHUNK_EDGE, NCHUNK_EDGE)], dst_v)

    for t in range(4):
        pltpu.async_copy(hp.at[src_v.at[t]], rows_v[t], gsem[t])

    def quad(q, carry):
        j = 4 * q
        sc = []
        for t in range(4):
            pltpu.make_async_copy(hp.at[src_v.at[j + t]], rows_v[t],
                                  gsem[t]).wait()
            sc.append(pltpu.async_copy(rows_v[t], acc.at[dst_v.at[j + t]],
                                       ssem[t], add=True))
        for t in range(4):
            sc[t].wait()
            pltpu.async_copy(hp.at[src_v.at[j + 4 + t]], rows_v[t], gsem[t])
        return carry

    lax.fori_loop(0, NCHUNK_EDGE // 4 - 1, quad, 0)
    jf = NCHUNK_EDGE - 4
    sc = []
    for t in range(4):
        pltpu.make_async_copy(hp.at[src_v.at[jf + t]], rows_v[t],
                              gsem[t]).wait()
        sc.append(pltpu.async_copy(rows_v[t], acc.at[dst_v.at[jf + t]],
                                   ssem[t], add=True))
    for t in range(4):
        sc[t].wait()
    plsc.subcore_barrier()
    for r in range(STRIPE // RB):
        pltpu.sync_copy(acc.at[pl.ds(row0 + r * RB, RB)], stage_v)
        pltpu.sync_copy(stage_v, out.at[c, pl.ds(row0 + r * RB, RB)])


def _inv_body(dp_ref, o_ref):
    d = dp_ref[0] + dp_ref[1]
    o_ref[...] = lax.rsqrt(jnp.maximum(d, 1.0))


_tc_inv = pl.pallas_call(
    _inv_body,
    out_shape=jax.ShapeDtypeStruct((2, NPAD), jnp.float32),
)

RBLK = 2000


def _mm0_body(x_ref, w_ref, o_ref):
    o_ref[...] = jnp.dot(x_ref[...], w_ref[...],
                         preferred_element_type=jnp.float32)


_tc_mm0 = pl.pallas_call(
    _mm0_body,
    grid=(N // RBLK,),
    in_specs=[
        pl.BlockSpec((RBLK, H), lambda i: (i, 0)),
        pl.BlockSpec((H, H), lambda i: (0, 0)),
    ],
    out_specs=pl.BlockSpec((RBLK, H), lambda i: (i, 0)),
    out_shape=jax.ShapeDtypeStruct((N, H), jnp.float32),
)


def _scale_body(x_ref, s_ref, o_ref):
    o_ref[...] = x_ref[...] * s_ref[...]


_tc_scale = pl.pallas_call(
    _scale_body,
    grid=(N // RBLK,),
    in_specs=[
        pl.BlockSpec((RBLK, H), lambda i: (i, 0)),
        pl.BlockSpec((RBLK, 1), lambda i: (i, 0)),
    ],
    out_specs=pl.BlockSpec((RBLK, H), lambda i: (i, 0)),
    out_shape=jax.ShapeDtypeStruct((N, H), jnp.float32),
)


def _mid_body(a_ref, ii_ref, b_ref, io_ref, w_ref, o_ref):
    a = jnp.concatenate([a_ref[0], a_ref[1]], axis=-1)
    h = jnp.maximum(a * ii_ref[...] + b_ref[...], 0.0)
    o_ref[...] = jnp.dot(h * io_ref[...], w_ref[...],
                         preferred_element_type=jnp.float32)


_tc_mid128 = pl.pallas_call(
    _mid_body,
    grid=(N // RBLK,),
    in_specs=[
        pl.BlockSpec((2, RBLK, HH), lambda i: (0, i, 0)),
        pl.BlockSpec((RBLK, 1), lambda i: (i, 0)),
        pl.BlockSpec((1, H), lambda i: (0, 0)),
        pl.BlockSpec((RBLK, 1), lambda i: (i, 0)),
        pl.BlockSpec((H, H), lambda i: (0, 0)),
    ],
    out_specs=pl.BlockSpec((RBLK, H), lambda i: (i, 0)),
    out_shape=jax.ShapeDtypeStruct((N, H), jnp.float32),
)


def _final_body(a_ref, ii_ref, b_ref, o_ref):
    a = a_ref[0] + a_ref[1]
    o_ref[...] = a * ii_ref[...] + b_ref[...]


_tc_final = pl.pallas_call(
    _final_body,
    grid=(N // RBLK,),
    in_specs=[
        pl.BlockSpec((2, RBLK, C), lambda i: (0, i, 0)),
        pl.BlockSpec((RBLK, 1), lambda i: (i, 0)),
        pl.BlockSpec((1, C), lambda i: (0, 0)),
    ],
    out_specs=pl.BlockSpec((RBLK, C), lambda i: (i, 0)),
    out_shape=jax.ShapeDtypeStruct((N, C), jnp.float32),
)


def kernel(in_feat, edge_index, W1, b1, W2, b2, W3, b3, W4, b4, W5, b5):
    src = edge_index[0]
    dst = edge_index[1]
    src2d = src.reshape(E // K, K)
    dst2d = dst.reshape(E // K, K)
    srcx = jnp.stack([2 * src, 2 * src + 1]).reshape(NC, E // K, K)
    zeros1 = jnp.zeros((STRIPE,), jnp.float32)
    ones_k = jnp.ones((K,), jnp.float32)
    zeros64 = jnp.zeros((RB, HH), jnp.float32)
    zeros16 = jnp.zeros((RB, C), jnp.float32)

    xw = _tc_mm0(in_feat, W1)
    degp = _sc_degrees(src2d, dst2d, zeros1, ones_k).reshape(2, 2, NPAD)
    invs = _tc_inv(degp)
    io = invs[0, :N][:, None]
    ii = invs[1, :N][:, None]

    hp = _tc_scale(xw, io)

    W5pad = jnp.pad(W5, ((0, 0), (0, H - C)))
    Ws = jnp.stack([W2, W3, W4, W5pad])
    bs = jnp.stack([b1, b2, b3, b4])[:, None, :]

    def step(h, wb):
        b, W = wb
        agg = _sc_agg128(h.reshape(2 * N, HH), srcx, dst2d, zeros64)
        return _tc_mid128(agg, ii, b, io, W), None

    hp5, _ = lax.scan(step, hp, (bs, Ws))
    agg = _sc_agg16(hp5[:, :C], src2d, dst2d, zeros16)
    return _tc_final(agg, ii, b5.reshape(1, C))


# speedup vs baseline: 1.0095x; 1.0095x over previous
import functools

import jax
import jax.numpy as jnp
from jax import lax
from jax.experimental import pallas as pl
from jax.experimental.pallas import tpu as pltpu
from jax.experimental.pallas import tpu_sc as plsc

N = 10000
E = 320000
H = 128
HH = H // 2
C = 16

NC = 2
NS = 16
NW = NC * NS
K = 125
NPAD = 10240
STRIPE = NPAD // NS
RB = 128

NCHUNK_CORE = E // NS // K
NCHUNK_EDGE = E // NW // K

_NOTILE = pltpu.CompilerParams(use_tc_tiling_on_sc=False)


def _sc_mesh():
    return plsc.VectorSubcoreMesh(core_axis_name="c", subcore_axis_name="s")


@functools.partial(
    pl.kernel,
    mesh=_sc_mesh(),
    out_type=jax.ShapeDtypeStruct((NC * 2 * NPAD,), jnp.float32),
    scratch_types=[
        pltpu.VMEM((NCHUNK_EDGE, K), jnp.int32),
        pltpu.VMEM((NCHUNK_EDGE, K), jnp.int32),
        pltpu.VMEM((K,), jnp.float32),
        pltpu.VMEM((STRIPE,), jnp.float32),
        pltpu.VMEM_SHARED((NPAD,), jnp.float32),
        pltpu.VMEM_SHARED((NPAD,), jnp.float32),
    ],
)
def _sc_degrees(src2d, dst2d, zeros1, ones_k, out, src_v, dst_v, ones_v,
                stage_v, deg_o, deg_i):
    c = lax.axis_index("c")
    s = lax.axis_index("s")
    wid = s * NC + c
    pltpu.sync_copy(zeros1, stage_v)
    pltpu.sync_copy(stage_v, deg_o.at[pl.ds(s * STRIPE, STRIPE)])
    pltpu.sync_copy(stage_v, deg_i.at[pl.ds(s * STRIPE, STRIPE)])
    plsc.subcore_barrier()
    pltpu.sync_copy(ones_k, ones_v)
    pltpu.sync_copy(src2d.at[pl.ds(wid * NCHUNK_EDGE, NCHUNK_EDGE)], src_v)
    pltpu.sync_copy(dst2d.at[pl.ds(wid * NCHUNK_EDGE, NCHUNK_EDGE)], dst_v)

    def chunk(j, carry):
        pltpu.sync_copy(ones_v, deg_o.at[src_v.at[j]], add=True)
        pltpu.sync_copy(ones_v, deg_i.at[dst_v.at[j]], add=True)
        return carry

    lax.fori_loop(0, NCHUNK_EDGE, chunk, 0)
    plsc.subcore_barrier()
    pltpu.sync_copy(deg_o.at[pl.ds(s * STRIPE, STRIPE)], stage_v)
    pltpu.sync_copy(stage_v, out.at[pl.ds((c * 2 + 0) * NPAD + s * STRIPE,
                                          STRIPE)])
    pltpu.sync_copy(deg_i.at[pl.ds(s * STRIPE, STRIPE)], stage_v)
    pltpu.sync_copy(stage_v, out.at[pl.ds((c * 2 + 1) * NPAD + s * STRIPE,
                                          STRIPE)])


@functools.partial(
    pl.kernel,
    mesh=_sc_mesh(),
    compiler_params=_NOTILE,
    out_type=jax.ShapeDtypeStruct((NC, NPAD, HH), jnp.float32),
    scratch_types=[
        pltpu.VMEM((NCHUNK_CORE, K), jnp.int32),
        pltpu.VMEM((NCHUNK_CORE, K), jnp.int32),
        [pltpu.VMEM((K, HH), jnp.float32)] * 4,
        pltpu.VMEM((RB, HH), jnp.float32),
        pltpu.VMEM_SHARED((NPAD, HH), jnp.float32),
        [pltpu.SemaphoreType.DMA] * 4,
        [pltpu.SemaphoreType.DMA] * 4,
    ],
)
def _sc_agg128(hp2, srcx, dst2d, zrows, out, src_v, dst_v, rows_v, stage_v,
               acc, gsem, ssem):
    c = lax.axis_index("c")
    s = lax.axis_index("s")
    row0 = s * STRIPE
    cp_s = pltpu.async_copy(srcx.at[c, pl.ds(s * NCHUNK_CORE, NCHUNK_CORE)],
                            src_v, gsem[0])
    cp_d = pltpu.async_copy(dst2d.at[pl.ds(s * NCHUNK_CORE, NCHUNK_CORE)],
                            dst_v, gsem[1])
    pltpu.sync_copy(zrows, stage_v)
    for r in range(STRIPE // RB):
        pltpu.sync_copy(stage_v, acc.at[pl.ds(row0 + r * RB, RB)])
    cp_s.wait()
    cp_d.wait()
    plsc.subcore_barrier()

    NB = 4
    for t in range(NB):
        pltpu.async_copy(hp2.at[src_v.at[t]], rows_v[t], gsem[t])

    def wave(q, carry):
        j = NB * q
        sc = []
        for t in range(NB):
            pltpu.make_async_copy(hp2.at[src_v.at[j + t]], rows_v[t],
                                  gsem[t]).wait()
            sc.append(pltpu.async_copy(rows_v[t], acc.at[dst_v.at[j + t]],
                                       ssem[t], add=True))
        for t in range(NB):
            sc[t].wait()
            pltpu.async_copy(hp2.at[src_v.at[j + NB + t]], rows_v[t], gsem[t])
        return carry

    lax.fori_loop(0, NCHUNK_CORE // NB - 1, wave, 0)
    jf = NCHUNK_CORE - NB
    sc = []
    for t in range(NB):
        pltpu.make_async_copy(hp2.at[src_v.at[jf + t]], rows_v[t],
                              gsem[t]).wait()
        sc.append(pltpu.async_copy(rows_v[t], acc.at[dst_v.at[jf + t]],
                                   ssem[t], add=True))
    for t in range(NB):
        sc[t].wait()
    plsc.subcore_barrier()
    for r in range(STRIPE // RB):
        pltpu.sync_copy(acc.at[pl.ds(row0 + r * RB, RB)], stage_v)
        pltpu.sync_copy(stage_v, out.at[c, pl.ds(row0 + r * RB, RB)])


@functools.partial(
    pl.kernel,
    mesh=_sc_mesh(),
    compiler_params=_NOTILE,
    out_type=jax.ShapeDtypeStruct((NC, NPAD, C), jnp.float32),
    scratch_types=[
        pltpu.VMEM((NCHUNK_EDGE, K), jnp.int32),
        pltpu.VMEM((NCHUNK_EDGE, K), jnp.int32),
        [pltpu.VMEM((K, C), jnp.float32)] * 4,
        pltpu.VMEM((RB, C), jnp.float32),
        pltpu.VMEM_SHARED((NPAD, C), jnp.float32),
        [pltpu.SemaphoreType.DMA] * 4,
        [pltpu.SemaphoreType.DMA] * 4,
    ],
)
def _sc_agg16(hp, src2d, dst2d, zrows, out, src_v, dst_v, rows_v, stage_v,
              acc, gsem, ssem):
    c = lax.axis_index("c")
    s = lax.axis_index("s")
    wid = s * NC + c
    row0 = s * STRIPE
    pltpu.sync_copy(zrows, stage_v)
    for r in range(STRIPE // RB):
        pltpu.sync_copy(stage_v, acc.at[pl.ds(row0 + r * RB, RB)])
    plsc.subcore_barrier()
    pltpu.sync_copy(src2d.at[pl.ds(wid * NCHUNK_EDGE, NCHUNK_EDGE)], src_v)
    pltpu.sync_copy(dst2d.at[pl.ds(wid * NCHUNK_EDGE, NCHUNK_EDGE)], dst_v)

    for t in range(4):
        pltpu.async_copy(hp.at[src_v.at[t]], rows_v[t], gsem[t])

    def quad(q, carry):
        j = 4 * q
        sc = []
        for t in range(4):
            pltpu.make_async_copy(hp.at[src_v.at[j + t]], rows_v[t],
                                  gsem[t]).wait()
            sc.append(pltpu.async_copy(rows_v[t], acc.at[dst_v.at[j + t]],
                                       ssem[t], add=True))
        for t in range(4):
            sc[t].wait()
            pltpu.async_copy(hp.at[src_v.at[j + 4 + t]], rows_v[t], gsem[t])
        return carry

    lax.fori_loop(0, NCHUNK_EDGE // 4 - 1, quad, 0)
    jf = NCHUNK_EDGE - 4
    sc = []
    for t in range(4):
        pltpu.make_async_copy(hp.at[src_v.at[jf + t]], rows_v[t],
                              gsem[t]).wait()
        sc.append(pltpu.async_copy(rows_v[t], acc.at[dst_v.at[jf + t]],
                                   ssem[t], add=True))
    for t in range(4):
        sc[t].wait()
    plsc.subcore_barrier()
    for r in range(STRIPE // RB):
        pltpu.sync_copy(acc.at[pl.ds(row0 + r * RB, RB)], stage_v)
        pltpu.sync_copy(stage_v, out.at[c, pl.ds(row0 + r * RB, RB)])


def _inv_body(dp_ref, o_ref):
    d = dp_ref[0] + dp_ref[1]
    o_ref[...] = lax.rsqrt(jnp.maximum(d, 1.0))


_tc_inv = pl.pallas_call(
    _inv_body,
    out_shape=jax.ShapeDtypeStruct((2, NPAD), jnp.float32),
)

RBLK = 2000


def _mm1_body(x_ref, s_ref, w_ref, o_ref):
    o_ref[...] = jnp.dot(x_ref[...] * s_ref[...], w_ref[...],
                         preferred_element_type=jnp.float32)


_tc_mm1 = pl.pallas_call(
    _mm1_body,
    grid=(N // RBLK,),
    in_specs=[
        pl.BlockSpec((RBLK, H), lambda i: (i, 0)),
        pl.BlockSpec((RBLK, 1), lambda i: (i, 0)),
        pl.BlockSpec((H, H), lambda i: (0, 0)),
    ],
    out_specs=pl.BlockSpec((RBLK, H), lambda i: (i, 0)),
    out_shape=jax.ShapeDtypeStruct((N, H), jnp.float32),
)


def _mid_body(a_ref, ii_ref, b_ref, io_ref, w_ref, o_ref):
    a = jnp.concatenate([a_ref[0], a_ref[1]], axis=-1)
    h = jnp.maximum(a * ii_ref[...] + b_ref[...], 0.0)
    o_ref[...] = jnp.dot(h * io_ref[...], w_ref[...],
                         preferred_element_type=jnp.float32)


_tc_mid128 = pl.pallas_call(
    _mid_body,
    grid=(N // RBLK,),
    in_specs=[
        pl.BlockSpec((2, RBLK, HH), lambda i: (0, i, 0)),
        pl.BlockSpec((RBLK, 1), lambda i: (i, 0)),
        pl.BlockSpec((1, H), lambda i: (0, 0)),
        pl.BlockSpec((RBLK, 1), lambda i: (i, 0)),
        pl.BlockSpec((H, H), lambda i: (0, 0)),
    ],
    out_specs=pl.BlockSpec((RBLK, H), lambda i: (i, 0)),
    out_shape=jax.ShapeDtypeStruct((N, H), jnp.float32),
)


def _final_body(a_ref, ii_ref, b_ref, o_ref):
    a = a_ref[0] + a_ref[1]
    o_ref[...] = a * ii_ref[...] + b_ref[...]


_tc_final = pl.pallas_call(
    _final_body,
    grid=(N // RBLK,),
    in_specs=[
        pl.BlockSpec((2, RBLK, C), lambda i: (0, i, 0)),
        pl.BlockSpec((RBLK, 1), lambda i: (i, 0)),
        pl.BlockSpec((1, C), lambda i: (0, 0)),
    ],
    out_specs=pl.BlockSpec((RBLK, C), lambda i: (i, 0)),
    out_shape=jax.ShapeDtypeStruct((N, C), jnp.float32),
)


def kernel(in_feat, edge_index, W1, b1, W2, b2, W3, b3, W4, b4, W5, b5):
    src = edge_index[0]
    dst = edge_index[1]
    src2d = src.reshape(E // K, K)
    dst2d = dst.reshape(E // K, K)
    srcx = jnp.stack([2 * src, 2 * src + 1]).reshape(NC, E // K, K)
    zeros1 = jnp.zeros((STRIPE,), jnp.float32)
    ones_k = jnp.ones((K,), jnp.float32)
    zeros64 = jnp.zeros((RB, HH), jnp.float32)
    zeros16 = jnp.zeros((RB, C), jnp.float32)

    degp = _sc_degrees(src2d, dst2d, zeros1, ones_k).reshape(2, 2, NPAD)
    invs = _tc_inv(degp)
    io = invs[0, :N][:, None]
    ii = invs[1, :N][:, None]

    hp = _tc_mm1(in_feat, io, W1)

    W5pad = jnp.pad(W5, ((0, 0), (0, H - C)))
    Ws = jnp.stack([W2, W3, W4, W5pad])
    bs = jnp.stack([b1, b2, b3, b4])[:, None, :]

    def step(h, wb):
        b, W = wb
        agg = _sc_agg128(h.reshape(2 * N, HH), srcx, dst2d, zeros64)
        return _tc_mid128(agg, ii, b, io, W), None

    hp5, _ = lax.scan(step, hp, (bs, Ws))
    agg = _sc_agg16(hp5[:, :C], src2d, dst2d, zeros16)
    return _tc_final(agg, ii, b5.reshape(1, C))


# speedup vs baseline: 1.0123x; 1.0027x over previous
import functools

import jax
import jax.numpy as jnp
from jax import lax
from jax.experimental import pallas as pl
from jax.experimental.pallas import tpu as pltpu
from jax.experimental.pallas import tpu_sc as plsc

N = 10000
E = 320000
H = 128
HH = H // 2
C = 16

NC = 2
NS = 16
NW = NC * NS
K = 125
NPAD = 10240
STRIPE = NPAD // NS
RB = 128

NCHUNK_CORE = E // NS // K
NCHUNK_EDGE = E // NW // K

_NOTILE = pltpu.CompilerParams(use_tc_tiling_on_sc=False)


def _sc_mesh():
    return plsc.VectorSubcoreMesh(core_axis_name="c", subcore_axis_name="s")


@functools.partial(
    pl.kernel,
    mesh=_sc_mesh(),
    out_type=jax.ShapeDtypeStruct((NC * 2 * NPAD,), jnp.float32),
    scratch_types=[
        pltpu.VMEM((NCHUNK_EDGE, K), jnp.int32),
        pltpu.VMEM((NCHUNK_EDGE, K), jnp.int32),
        pltpu.VMEM((K,), jnp.float32),
        pltpu.VMEM((STRIPE,), jnp.float32),
        pltpu.VMEM_SHARED((NPAD,), jnp.float32),
        pltpu.VMEM_SHARED((NPAD,), jnp.float32),
    ],
)
def _sc_degrees(src2d, dst2d, zeros1, ones_k, out, src_v, dst_v, ones_v,
                stage_v, deg_o, deg_i):
    c = lax.axis_index("c")
    s = lax.axis_index("s")
    wid = s * NC + c
    pltpu.sync_copy(zeros1, stage_v)
    pltpu.sync_copy(stage_v, deg_o.at[pl.ds(s * STRIPE, STRIPE)])
    pltpu.sync_copy(stage_v, deg_i.at[pl.ds(s * STRIPE, STRIPE)])
    plsc.subcore_barrier()
    pltpu.sync_copy(ones_k, ones_v)
    pltpu.sync_copy(src2d.at[pl.ds(wid * NCHUNK_EDGE, NCHUNK_EDGE)], src_v)
    pltpu.sync_copy(dst2d.at[pl.ds(wid * NCHUNK_EDGE, NCHUNK_EDGE)], dst_v)

    def chunk(j, carry):
        pltpu.sync_copy(ones_v, deg_o.at[src_v.at[j]], add=True)
        pltpu.sync_copy(ones_v, deg_i.at[dst_v.at[j]], add=True)
        return carry

    lax.fori_loop(0, NCHUNK_EDGE, chunk, 0)
    plsc.subcore_barrier()
    pltpu.sync_copy(deg_o.at[pl.ds(s * STRIPE, STRIPE)], stage_v)
    pltpu.sync_copy(stage_v, out.at[pl.ds((c * 2 + 0) * NPAD + s * STRIPE,
                                          STRIPE)])
    pltpu.sync_copy(deg_i.at[pl.ds(s * STRIPE, STRIPE)], stage_v)
    pltpu.sync_copy(stage_v, out.at[pl.ds((c * 2 + 1) * NPAD + s * STRIPE,
                                          STRIPE)])


@functools.partial(
    pl.kernel,
    mesh=_sc_mesh(),
    compiler_params=_NOTILE,
    out_type=jax.ShapeDtypeStruct((NC, NPAD, HH), jnp.float32),
    scratch_types=[
        pltpu.VMEM((NCHUNK_CORE, K), jnp.int32),
        pltpu.VMEM((NCHUNK_CORE, K), jnp.int32),
        [pltpu.VMEM((K, HH), jnp.float32)] * 4,
        pltpu.VMEM((RB, HH), jnp.float32),
        pltpu.VMEM_SHARED((NPAD, HH), jnp.float32),
        [pltpu.SemaphoreType.DMA] * 4,
        [pltpu.SemaphoreType.DMA] * 4,
    ],
)
def _sc_agg128(hp2, srcx, dst2d, zrows, out, src_v, dst_v, rows_v, stage_v,
               acc, gsem, ssem):
    c = lax.axis_index("c")
    s = lax.axis_index("s")
    row0 = s * STRIPE
    cp_s = pltpu.async_copy(srcx.at[c, pl.ds(s * NCHUNK_CORE, NCHUNK_CORE)],
                            src_v, gsem[0])
    cp_d = pltpu.async_copy(dst2d.at[pl.ds(s * NCHUNK_CORE, NCHUNK_CORE)],
                            dst_v, gsem[1])
    pltpu.sync_copy(zrows, stage_v)
    for r in range(STRIPE // RB):
        pltpu.sync_copy(stage_v, acc.at[pl.ds(row0 + r * RB, RB)])
    cp_s.wait()
    cp_d.wait()
    plsc.subcore_barrier()

    NB = 4
    for t in range(NB):
        pltpu.async_copy(hp2.at[src_v.at[t]], rows_v[t], gsem[t])

    def wave(q, carry):
        j = NB * q
        sc = []
        for t in range(NB):
            pltpu.make_async_copy(hp2.at[src_v.at[j + t]], rows_v[t],
                                  gsem[t]).wait()
            sc.append(pltpu.async_copy(rows_v[t], acc.at[dst_v.at[j + t]],
                                       ssem[t], add=True))
        for t in range(NB):
            sc[t].wait()
            pltpu.async_copy(hp2.at[src_v.at[j + NB + t]], rows_v[t], gsem[t])
        return carry

    lax.fori_loop(0, NCHUNK_CORE // NB - 1, wave, 0)
    jf = NCHUNK_CORE - NB
    sc = []
    for t in range(NB):
        pltpu.make_async_copy(hp2.at[src_v.at[jf + t]], rows_v[t],
                              gsem[t]).wait()
        sc.append(pltpu.async_copy(rows_v[t], acc.at[dst_v.at[jf + t]],
                                   ssem[t], add=True))
    for t in range(NB):
        sc[t].wait()
    plsc.subcore_barrier()
    for r in range(STRIPE // RB):
        pltpu.sync_copy(acc.at[pl.ds(row0 + r * RB, RB)], stage_v)
        pltpu.sync_copy(stage_v, out.at[c, pl.ds(row0 + r * RB, RB)])


@functools.partial(
    pl.kernel,
    mesh=_sc_mesh(),
    compiler_params=_NOTILE,
    out_type=jax.ShapeDtypeStruct((NC, NPAD, C), jnp.float32),
    scratch_types=[
        pltpu.VMEM((NCHUNK_EDGE, K), jnp.int32),
        pltpu.VMEM((NCHUNK_EDGE, K), jnp.int32),
        [pltpu.VMEM((K, C), jnp.float32)] * 4,
        pltpu.VMEM((RB, C), jnp.float32),
        pltpu.VMEM_SHARED((NPAD, C), jnp.float32),
        [pltpu.SemaphoreType.DMA] * 4,
        [pltpu.SemaphoreType.DMA] * 4,
    ],
)
def _sc_agg16(hp, src2d, dst2d, zrows, out, src_v, dst_v, rows_v, stage_v,
              acc, gsem, ssem):
    c = lax.axis_index("c")
    s = lax.axis_index("s")
    wid = s * NC + c
    row0 = s * STRIPE
    cp_s = pltpu.async_copy(src2d.at[pl.ds(wid * NCHUNK_EDGE, NCHUNK_EDGE)],
                            src_v, gsem[0])
    cp_d = pltpu.async_copy(dst2d.at[pl.ds(wid * NCHUNK_EDGE, NCHUNK_EDGE)],
                            dst_v, gsem[1])
    pltpu.sync_copy(zrows, stage_v)
    for r in range(STRIPE // RB):
        pltpu.sync_copy(stage_v, acc.at[pl.ds(row0 + r * RB, RB)])
    cp_s.wait()
    cp_d.wait()
    plsc.subcore_barrier()

    for t in range(4):
        pltpu.async_copy(hp.at[src_v.at[t]], rows_v[t], gsem[t])

    def quad(q, carry):
        j = 4 * q
        sc = []
        for t in range(4):
            pltpu.make_async_copy(hp.at[src_v.at[j + t]], rows_v[t],
                                  gsem[t]).wait()
            sc.append(pltpu.async_copy(rows_v[t], acc.at[dst_v.at[j + t]],
                                       ssem[t], add=True))
        for t in range(4):
            sc[t].wait()
            pltpu.async_copy(hp.at[src_v.at[j + 4 + t]], rows_v[t], gsem[t])
        return carry

    lax.fori_loop(0, NCHUNK_EDGE // 4 - 1, quad, 0)
    jf = NCHUNK_EDGE - 4
    sc = []
    for t in range(4):
        pltpu.make_async_copy(hp.at[src_v.at[jf + t]], rows_v[t],
                              gsem[t]).wait()
        sc.append(pltpu.async_copy(rows_v[t], acc.at[dst_v.at[jf + t]],
                                   ssem[t], add=True))
    for t in range(4):
        sc[t].wait()
    plsc.subcore_barrier()
    for r in range(STRIPE // RB):
        pltpu.sync_copy(acc.at[pl.ds(row0 + r * RB, RB)], stage_v)
        pltpu.sync_copy(stage_v, out.at[c, pl.ds(row0 + r * RB, RB)])


def _inv_body(dp_ref, o_ref):
    d = dp_ref[0] + dp_ref[1]
    o_ref[...] = lax.rsqrt(jnp.maximum(d, 1.0))


_tc_inv = pl.pallas_call(
    _inv_body,
    out_shape=jax.ShapeDtypeStruct((2, NPAD), jnp.float32),
)

RBLK = 2000


def _mm1_body(x_ref, s_ref, w_ref, o_ref):
    o_ref[...] = jnp.dot(x_ref[...] * s_ref[...], w_ref[...],
                         preferred_element_type=jnp.float32)


_tc_mm1 = pl.pallas_call(
    _mm1_body,
    grid=(N // RBLK,),
    in_specs=[
        pl.BlockSpec((RBLK, H), lambda i: (i, 0)),
        pl.BlockSpec((RBLK, 1), lambda i: (i, 0)),
        pl.BlockSpec((H, H), lambda i: (0, 0)),
    ],
    out_specs=pl.BlockSpec((RBLK, H), lambda i: (i, 0)),
    out_shape=jax.ShapeDtypeStruct((N, H), jnp.float32),
)


def _mid_body(a_ref, ii_ref, b_ref, io_ref, w_ref, o_ref):
    a = jnp.concatenate([a_ref[0], a_ref[1]], axis=-1)
    h = jnp.maximum(a * ii_ref[...] + b_ref[...], 0.0)
    o_ref[...] = jnp.dot(h * io_ref[...], w_ref[...],
                         preferred_element_type=jnp.float32)


_tc_mid128 = pl.pallas_call(
    _mid_body,
    grid=(N // RBLK,),
    in_specs=[
        pl.BlockSpec((2, RBLK, HH), lambda i: (0, i, 0)),
        pl.BlockSpec((RBLK, 1), lambda i: (i, 0)),
        pl.BlockSpec((1, H), lambda i: (0, 0)),
        pl.BlockSpec((RBLK, 1), lambda i: (i, 0)),
        pl.BlockSpec((H, H), lambda i: (0, 0)),
    ],
    out_specs=pl.BlockSpec((RBLK, H), lambda i: (i, 0)),
    out_shape=jax.ShapeDtypeStruct((N, H), jnp.float32),
)


def _final_body(a_ref, ii_ref, b_ref, o_ref):
    a = a_ref[0] + a_ref[1]
    o_ref[...] = a * ii_ref[...] + b_ref[...]


_tc_final = pl.pallas_call(
    _final_body,
    grid=(N // RBLK,),
    in_specs=[
        pl.BlockSpec((2, RBLK, C), lambda i: (0, i, 0)),
        pl.BlockSpec((RBLK, 1), lambda i: (i, 0)),
        pl.BlockSpec((1, C), lambda i: (0, 0)),
    ],
    out_specs=pl.BlockSpec((RBLK, C), lambda i: (i, 0)),
    out_shape=jax.ShapeDtypeStruct((N, C), jnp.float32),
)


def kernel(in_feat, edge_index, W1, b1, W2, b2, W3, b3, W4, b4, W5, b5):
    src = edge_index[0]
    dst = edge_index[1]
    src2d = src.reshape(E // K, K)
    dst2d = dst.reshape(E // K, K)
    srcx = jnp.stack([2 * src, 2 * src + 1]).reshape(NC, E // K, K)
    zeros1 = jnp.zeros((STRIPE,), jnp.float32)
    ones_k = jnp.ones((K,), jnp.float32)
    zeros64 = jnp.zeros((RB, HH), jnp.float32)
    zeros16 = jnp.zeros((RB, C), jnp.float32)

    degp = _sc_degrees(src2d, dst2d, zeros1, ones_k).reshape(2, 2, NPAD)
    invs = _tc_inv(degp)
    io = invs[0, :N][:, None]
    ii = invs[1, :N][:, None]

    hp = _tc_mm1(in_feat, io, W1)

    W5pad = jnp.pad(W5, ((0, 0), (0, H - C)))
    Ws = jnp.stack([W2, W3, W4, W5pad])
    bs = jnp.stack([b1, b2, b3, b4])[:, None, :]

    def step(h, wb):
        b, W = wb
        agg = _sc_agg128(h.reshape(2 * N, HH), srcx, dst2d, zeros64)
        return _tc_mid128(agg, ii, b, io, W), None

    hp5, _ = lax.scan(step, hp, (bs, Ws))
    agg = _sc_agg16(hp5[:, :C], src2d, dst2d, zeros16)
    return _tc_final(agg, ii, b5.reshape(1, C))


# speedup vs baseline: 1.0157x; 1.0034x over previous
import functools

import jax
import jax.numpy as jnp
from jax import lax
from jax.experimental import pallas as pl
from jax.experimental.pallas import tpu as pltpu
from jax.experimental.pallas import tpu_sc as plsc

N = 10000
E = 320000
H = 128
HH = H // 2
C = 16

NC = 2
NS = 16
NW = NC * NS
K = 125
NPAD = 10240
STRIPE = NPAD // NS
RB = 128

NCHUNK_CORE = E // NS // K
NCHUNK_EDGE = E // NW // K

_NOTILE = pltpu.CompilerParams(use_tc_tiling_on_sc=False)


def _sc_mesh():
    return plsc.VectorSubcoreMesh(core_axis_name="c", subcore_axis_name="s")


@functools.partial(
    pl.kernel,
    mesh=_sc_mesh(),
    out_type=jax.ShapeDtypeStruct((NC * 2 * NPAD,), jnp.float32),
    scratch_types=[
        pltpu.VMEM((NCHUNK_EDGE, K), jnp.int32),
        pltpu.VMEM((NCHUNK_EDGE, K), jnp.int32),
        pltpu.VMEM((K,), jnp.float32),
        pltpu.VMEM((STRIPE,), jnp.float32),
        pltpu.VMEM_SHARED((NPAD,), jnp.float32),
        pltpu.VMEM_SHARED((NPAD,), jnp.float32),
        [pltpu.SemaphoreType.DMA] * 2,
    ],
)
def _sc_degrees(src2d, dst2d, zeros1, ones_k, out, src_v, dst_v, ones_v,
                stage_v, deg_o, deg_i, isem):
    c = lax.axis_index("c")
    s = lax.axis_index("s")
    wid = s * NC + c
    cp_s = pltpu.async_copy(src2d.at[pl.ds(wid * NCHUNK_EDGE, NCHUNK_EDGE)],
                            src_v, isem[0])
    cp_d = pltpu.async_copy(dst2d.at[pl.ds(wid * NCHUNK_EDGE, NCHUNK_EDGE)],
                            dst_v, isem[1])
    pltpu.sync_copy(zeros1, stage_v)
    pltpu.sync_copy(stage_v, deg_o.at[pl.ds(s * STRIPE, STRIPE)])
    pltpu.sync_copy(stage_v, deg_i.at[pl.ds(s * STRIPE, STRIPE)])
    pltpu.sync_copy(ones_k, ones_v)
    cp_s.wait()
    cp_d.wait()
    plsc.subcore_barrier()

    def chunk(j, carry):
        pltpu.sync_copy(ones_v, deg_o.at[src_v.at[j]], add=True)
        pltpu.sync_copy(ones_v, deg_i.at[dst_v.at[j]], add=True)
        return carry

    lax.fori_loop(0, NCHUNK_EDGE, chunk, 0)
    plsc.subcore_barrier()
    pltpu.sync_copy(deg_o.at[pl.ds(s * STRIPE, STRIPE)], stage_v)
    pltpu.sync_copy(stage_v, out.at[pl.ds((c * 2 + 0) * NPAD + s * STRIPE,
                                          STRIPE)])
    pltpu.sync_copy(deg_i.at[pl.ds(s * STRIPE, STRIPE)], stage_v)
    pltpu.sync_copy(stage_v, out.at[pl.ds((c * 2 + 1) * NPAD + s * STRIPE,
                                          STRIPE)])


@functools.partial(
    pl.kernel,
    mesh=_sc_mesh(),
    compiler_params=_NOTILE,
    out_type=jax.ShapeDtypeStruct((NC, NPAD, HH), jnp.float32),
    scratch_types=[
        pltpu.VMEM((NCHUNK_CORE, K), jnp.int32),
        pltpu.VMEM((NCHUNK_CORE, K), jnp.int32),
        [pltpu.VMEM((K, HH), jnp.float32)] * 4,
        pltpu.VMEM((RB, HH), jnp.float32),
        pltpu.VMEM_SHARED((NPAD, HH), jnp.float32),
        [pltpu.SemaphoreType.DMA] * 4,
        [pltpu.SemaphoreType.DMA] * 4,
    ],
)
def _sc_agg128(hp2, srcx, dst2d, zrows, out, src_v, dst_v, rows_v, stage_v,
               acc, gsem, ssem):
    c = lax.axis_index("c")
    s = lax.axis_index("s")
    row0 = s * STRIPE
    cp_s = pltpu.async_copy(srcx.at[c, pl.ds(s * NCHUNK_CORE, NCHUNK_CORE)],
                            src_v, gsem[0])
    cp_d = pltpu.async_copy(dst2d.at[pl.ds(s * NCHUNK_CORE, NCHUNK_CORE)],
                            dst_v, gsem[1])
    pltpu.sync_copy(zrows, stage_v)
    for r in range(STRIPE // RB):
        pltpu.sync_copy(stage_v, acc.at[pl.ds(row0 + r * RB, RB)])
    cp_s.wait()
    cp_d.wait()
    plsc.subcore_barrier()

    NB = 4
    for t in range(NB):
        pltpu.async_copy(hp2.at[src_v.at[t]], rows_v[t], gsem[t])

    def wave(q, carry):
        j = NB * q
        sc = []
        for t in range(NB):
            pltpu.make_async_copy(hp2.at[src_v.at[j + t]], rows_v[t],
                                  gsem[t]).wait()
            sc.append(pltpu.async_copy(rows_v[t], acc.at[dst_v.at[j + t]],
                                       ssem[t], add=True))
        for t in range(NB):
            sc[t].wait()
            pltpu.async_copy(hp2.at[src_v.at[j + NB + t]], rows_v[t], gsem[t])
        return carry

    lax.fori_loop(0, NCHUNK_CORE // NB - 1, wave, 0)
    jf = NCHUNK_CORE - NB
    sc = []
    for t in range(NB):
        pltpu.make_async_copy(hp2.at[src_v.at[jf + t]], rows_v[t],
                              gsem[t]).wait()
        sc.append(pltpu.async_copy(rows_v[t], acc.at[dst_v.at[jf + t]],
                                   ssem[t], add=True))
    for t in range(NB):
        sc[t].wait()
    plsc.subcore_barrier()
    for r in range(STRIPE // RB):
        pltpu.sync_copy(acc.at[pl.ds(row0 + r * RB, RB)], stage_v)
        pltpu.sync_copy(stage_v, out.at[c, pl.ds(row0 + r * RB, RB)])


@functools.partial(
    pl.kernel,
    mesh=_sc_mesh(),
    compiler_params=_NOTILE,
    out_type=jax.ShapeDtypeStruct((NC, NPAD, C), jnp.float32),
    scratch_types=[
        pltpu.VMEM((NCHUNK_EDGE, K), jnp.int32),
        pltpu.VMEM((NCHUNK_EDGE, K), jnp.int32),
        [pltpu.VMEM((K, C), jnp.float32)] * 4,
        pltpu.VMEM((RB, C), jnp.float32),
        pltpu.VMEM_SHARED((NPAD, C), jnp.float32),
        [pltpu.SemaphoreType.DMA] * 4,
        [pltpu.SemaphoreType.DMA] * 4,
    ],
)
def _sc_agg16(hp, src2d, dst2d, zrows, out, src_v, dst_v, rows_v, stage_v,
              acc, gsem, ssem):
    c = lax.axis_index("c")
    s = lax.axis_index("s")
    wid = s * NC + c
    row0 = s * STRIPE
    cp_s = pltpu.async_copy(src2d.at[pl.ds(wid * NCHUNK_EDGE, NCHUNK_EDGE)],
                            src_v, gsem[0])
    cp_d = pltpu.async_copy(dst2d.at[pl.ds(wid * NCHUNK_EDGE, NCHUNK_EDGE)],
                            dst_v, gsem[1])
    pltpu.sync_copy(zrows, stage_v)
    for r in range(STRIPE // RB):
        pltpu.sync_copy(stage_v, acc.at[pl.ds(row0 + r * RB, RB)])
    cp_s.wait()
    cp_d.wait()
    plsc.subcore_barrier()

    for t in range(4):
        pltpu.async_copy(hp.at[src_v.at[t]], rows_v[t], gsem[t])

    def quad(q, carry):
        j = 4 * q
        sc = []
        for t in range(4):
            pltpu.make_async_copy(hp.at[src_v.at[j + t]], rows_v[t],
                                  gsem[t]).wait()
            sc.append(pltpu.async_copy(rows_v[t], acc.at[dst_v.at[j + t]],
                                       ssem[t], add=True))
        for t in range(4):
            sc[t].wait()
            pltpu.async_copy(hp.at[src_v.at[j + 4 + t]], rows_v[t], gsem[t])
        return carry

    lax.fori_loop(0, NCHUNK_EDGE // 4 - 1, quad, 0)
    jf = NCHUNK_EDGE - 4
    sc = []
    for t in range(4):
        pltpu.make_async_copy(hp.at[src_v.at[jf + t]], rows_v[t],
                              gsem[t]).wait()
        sc.append(pltpu.async_copy(rows_v[t], acc.at[dst_v.at[jf + t]],
                                   ssem[t], add=True))
    for t in range(4):
        sc[t].wait()
    plsc.subcore_barrier()
    for r in range(STRIPE // RB):
        pltpu.sync_copy(acc.at[pl.ds(row0 + r * RB, RB)], stage_v)
        pltpu.sync_copy(stage_v, out.at[c, pl.ds(row0 + r * RB, RB)])


def _inv_body(dp_ref, o_ref):
    d = dp_ref[0] + dp_ref[1]
    o_ref[...] = lax.rsqrt(jnp.maximum(d, 1.0))


_tc_inv = pl.pallas_call(
    _inv_body,
    out_shape=jax.ShapeDtypeStruct((2, NPAD), jnp.float32),
)

RBLK = 2000


def _mm1_body(x_ref, s_ref, w_ref, o_ref):
    o_ref[...] = jnp.dot(x_ref[...] * s_ref[...], w_ref[...],
                         preferred_element_type=jnp.float32)


_tc_mm1 = pl.pallas_call(
    _mm1_body,
    grid=(N // RBLK,),
    in_specs=[
        pl.BlockSpec((RBLK, H), lambda i: (i, 0)),
        pl.BlockSpec((RBLK, 1), lambda i: (i, 0)),
        pl.BlockSpec((H, H), lambda i: (0, 0)),
    ],
    out_specs=pl.BlockSpec((RBLK, H), lambda i: (i, 0)),
    out_shape=jax.ShapeDtypeStruct((N, H), jnp.float32),
)


def _mid_body(a_ref, ii_ref, b_ref, io_ref, w_ref, o_ref):
    a = jnp.concatenate([a_ref[0], a_ref[1]], axis=-1)
    h = jnp.maximum(a * ii_ref[...] + b_ref[...], 0.0)
    o_ref[...] = jnp.dot(h * io_ref[...], w_ref[...],
                         preferred_element_type=jnp.float32)


_tc_mid128 = pl.pallas_call(
    _mid_body,
    grid=(N // RBLK,),
    in_specs=[
        pl.BlockSpec((2, RBLK, HH), lambda i: (0, i, 0)),
        pl.BlockSpec((RBLK, 1), lambda i: (i, 0)),
        pl.BlockSpec((1, H), lambda i: (0, 0)),
        pl.BlockSpec((RBLK, 1), lambda i: (i, 0)),
        pl.BlockSpec((H, H), lambda i: (0, 0)),
    ],
    out_specs=pl.BlockSpec((RBLK, H), lambda i: (i, 0)),
    out_shape=jax.ShapeDtypeStruct((N, H), jnp.float32),
)


def _final_body(a_ref, ii_ref, b_ref, o_ref):
    a = a_ref[0] + a_ref[1]
    o_ref[...] = a * ii_ref[...] + b_ref[...]


_tc_final = pl.pallas_call(
    _final_body,
    grid=(N // RBLK,),
    in_specs=[
        pl.BlockSpec((2, RBLK, C), lambda i: (0, i, 0)),
        pl.BlockSpec((RBLK, 1), lambda i: (i, 0)),
        pl.BlockSpec((1, C), lambda i: (0, 0)),
    ],
    out_specs=pl.BlockSpec((RBLK, C), lambda i: (i, 0)),
    out_shape=jax.ShapeDtypeStruct((N, C), jnp.float32),
)


def kernel(in_feat, edge_index, W1, b1, W2, b2, W3, b3, W4, b4, W5, b5):
    src = edge_index[0]
    dst = edge_index[1]
    src2d = src.reshape(E // K, K)
    dst2d = dst.reshape(E // K, K)
    srcx = jnp.stack([2 * src, 2 * src + 1]).reshape(NC, E // K, K)
    zeros1 = jnp.zeros((STRIPE,), jnp.float32)
    ones_k = jnp.ones((K,), jnp.float32)
    zeros64 = jnp.zeros((RB, HH), jnp.float32)
    zeros16 = jnp.zeros((RB, C), jnp.float32)

    degp = _sc_degrees(src2d, dst2d, zeros1, ones_k).reshape(2, 2, NPAD)
    invs = _tc_inv(degp)
    io = invs[0, :N][:, None]
    ii = invs[1, :N][:, None]

    hp = _tc_mm1(in_feat, io, W1)

    W5pad = jnp.pad(W5, ((0, 0), (0, H - C)))
    Ws = jnp.stack([W2, W3, W4, W5pad])
    bs = jnp.stack([b1, b2, b3, b4])[:, None, :]

    def step(h, wb):
        b, W = wb
        agg = _sc_agg128(h.reshape(2 * N, HH), srcx, dst2d, zeros64)
        return _tc_mid128(agg, ii, b, io, W), None

    hp5, _ = lax.scan(step, hp, (bs, Ws))
    agg = _sc_agg16(hp5[:, :C], src2d, dst2d, zeros16)
    return _tc_final(agg, ii, b5.reshape(1, C))
